# baseline (speedup 1.0000x reference)
import functools

import jax
import jax.numpy as jnp
from jax import lax
from jax.experimental import pallas as pl
from jax.experimental.pallas import tpu as pltpu
from jax.experimental.pallas import tpu_sc as plsc

_N = 2048
_D = 128
_L = 100
_LP = 128
_NV = _N // 16
_NC = 2
_NS = 16
_NW = _NC * _NS
_IPW = (8 * _L) // _NW


def _proj_kernel(x_ref, y_ref, th_ref, px_ref, py_ref):
    th = th_ref[0]
    norm2 = jnp.sum(th * th, axis=1, keepdims=True)
    tn = th / jnp.sqrt(norm2)
    dn = (((1,), (1,)), ((), ()))
    px_ref[0] = lax.dot_general(tn, x_ref[0], dn,
                                preferred_element_type=jnp.float32)
    py_ref[0] = lax.dot_general(tn, y_ref[0], dn,
                                preferred_element_type=jnp.float32)


def _vreg_sort_pass(ref, K):
    def body(v, c):
        x = ref[pl.ds(v * 16, 16)]
        asc = (v & K) == 0
        t = jnp.where(asc, x, -x)
        s = jnp.sort(t)
        ref[pl.ds(v * 16, 16)] = jnp.where(asc, s, -s)
        return c
    lax.fori_loop(0, _NV, body, 0, unroll=2)


def _sort2048(ref):
    _vreg_sort_pass(ref, 1)
    K = 2
    while K <= _NV:
        J = K // 2
        while J >= 1:
            def body(p, c, J=J, K=K):
                v = (p // J) * (2 * J) + (p % J)
                w = v + J
                a = ref[pl.ds(v * 16, 16)]
                b = ref[pl.ds(w * 16, 16)]
                asc = (v & K) == 0
                mn = jnp.minimum(a, b)
                mx = jnp.maximum(a, b)
                ref[pl.ds(v * 16, 16)] = jnp.where(asc, mn, mx)
                ref[pl.ds(w * 16, 16)] = jnp.where(asc, mx, mn)
                return c
            lax.fori_loop(0, _NV // 2, body, 0, unroll=4)
            J //= 2
        _vreg_sort_pass(ref, K)
        K *= 2


def _sc_sort_ssd(px_hbm, py_hbm, out_hbm, vx, vy, vacc):
    wid = lax.axis_index("s") * _NC + lax.axis_index("c")
    b = wid // 4
    l0 = (wid % 4) * _IPW

    def item_body(i, acc):
        row = b * _LP + l0 + i
        pltpu.sync_copy(px_hbm.at[row], vx)
        pltpu.sync_copy(py_hbm.at[row], vy)
        _sort2048(vx)
        _sort2048(vy)

        def diff_body(v, a):
            d = vx[pl.ds(v * 16, 16)] - vy[pl.ds(v * 16, 16)]
            return a + d * d
        return lax.fori_loop(0, _NV, diff_body, acc, unroll=4)

    acc = lax.fori_loop(0, _IPW, item_body, jnp.zeros((16,), jnp.float32))
    vacc[...] = acc
    pltpu.sync_copy(vacc, out_hbm.at[wid])


def _final_kernel(part_ref, out_ref):
    v = part_ref[...]
    tot = jnp.sum(v, axis=1, keepdims=True)
    sw = jnp.sqrt(tot / (_N * _L))
    out_ref[0, 0] = jnp.sum(sw) / v.shape[0]


def kernel(X, Y, Pxy, Pyx, thetas):
    B = X.shape[0]
    th_p = jnp.pad(thetas, ((0, 0), (0, _LP - _L), (0, 0)))

    px, py = pl.pallas_call(
        _proj_kernel,
        grid=(B,),
        in_specs=[
            pl.BlockSpec((1, _N, _D), lambda b: (b, 0, 0)),
            pl.BlockSpec((1, _N, _D), lambda b: (b, 0, 0)),
            pl.BlockSpec((1, _LP, _D), lambda b: (b, 0, 0)),
        ],
        out_specs=[
            pl.BlockSpec((1, _LP, _N), lambda b: (b, 0, 0)),
            pl.BlockSpec((1, _LP, _N), lambda b: (b, 0, 0)),
        ],
        out_shape=[
            jax.ShapeDtypeStruct((B, _LP, _N), jnp.float32),
            jax.ShapeDtypeStruct((B, _LP, _N), jnp.float32),
        ],
    )(X, Y, th_p)

    mesh = plsc.VectorSubcoreMesh(
        core_axis_name="c", subcore_axis_name="s",
        num_cores=_NC, num_subcores=_NS)
    sc_call = pl.kernel(
        _sc_sort_ssd,
        mesh=mesh,
        out_type=jax.ShapeDtypeStruct((_NW, 16), jnp.float32),
        compiler_params=pltpu.CompilerParams(needs_layout_passes=False),
        scratch_types=[
            pltpu.VMEM((_N,), jnp.float32),
            pltpu.VMEM((_N,), jnp.float32),
            pltpu.VMEM((16,), jnp.float32),
        ],
    )
    part = sc_call(px.reshape(B * _LP, _N), py.reshape(B * _LP, _N))

    out = pl.pallas_call(
        _final_kernel,
        in_specs=[pl.BlockSpec((B, 4 * 16), lambda: (0, 0))],
        out_specs=pl.BlockSpec(memory_space=pltpu.SMEM),
        out_shape=jax.ShapeDtypeStruct((1, 1), jnp.float32),
    )(part.reshape(B, 4 * 16))

    return out[0, 0]


# speedup vs baseline: 1.2708x; 1.2708x over previous
import functools

import jax
import jax.numpy as jnp
from jax import lax
from jax.experimental import pallas as pl
from jax.experimental.pallas import tpu as pltpu
from jax.experimental.pallas import tpu_sc as plsc

_N = 2048
_D = 128
_L = 100
_LP = 128
_NV = _N // 16
_NC = 2
_NS = 16
_NW = _NC * _NS
_IPW = (8 * _L) // _NW


def _proj_kernel(x_ref, y_ref, th_ref, px_ref, py_ref):
    th = th_ref[0]
    norm2 = jnp.sum(th * th, axis=1, keepdims=True)
    tn = th / jnp.sqrt(norm2)
    dn = (((1,), (1,)), ((), ()))
    px_ref[0] = lax.dot_general(tn, x_ref[0], dn,
                                preferred_element_type=jnp.float32)
    py_ref[0] = lax.dot_general(tn, y_ref[0], dn,
                                preferred_element_type=jnp.float32)


def _vreg_sort_pass(ref, K):
    if K >= _NV:
        def body(v, c):
            x = ref[pl.ds(v * 16, 16)]
            s, _ = plsc.sort_key_val(x, x)
            ref[pl.ds(v * 16, 16)] = s
            return c
        lax.fori_loop(0, _NV, body, 0, unroll=8)
        return

    def mk_body(off, desc):
        def body(q, c):
            v = (q // K) * (2 * K) + (q % K) + off
            x = ref[pl.ds(v * 16, 16)]
            s, _ = plsc.sort_key_val(x, x, descending=desc)
            ref[pl.ds(v * 16, 16)] = s
            return c
        return body
    lax.fori_loop(0, _NV // 2, mk_body(0, False), 0, unroll=8)
    lax.fori_loop(0, _NV // 2, mk_body(K, True), 0, unroll=8)


def _pair_stage(ref, J, K):
    H = K // 2

    def mk_body(off, desc):
        def body(q, c):
            if H >= _NV // 2:
                p = q
            else:
                p = (q // H) * (2 * H) + (q % H) + off
            v = (p // J) * (2 * J) + (p % J)
            w = v + J
            a = ref[pl.ds(v * 16, 16)]
            b = ref[pl.ds(w * 16, 16)]
            mn = jnp.minimum(a, b)
            mx = jnp.maximum(a, b)
            if desc:
                mn, mx = mx, mn
            ref[pl.ds(v * 16, 16)] = mn
            ref[pl.ds(w * 16, 16)] = mx
            return c
        return body

    if H >= _NV // 2:
        lax.fori_loop(0, _NV // 2, mk_body(0, False), 0, unroll=8)
    else:
        lax.fori_loop(0, _NV // 4, mk_body(0, False), 0, unroll=8)
        lax.fori_loop(0, _NV // 4, mk_body(H, True), 0, unroll=8)


def _sort2048(ref):
    _vreg_sort_pass(ref, 1)
    K = 2
    while K <= _NV:
        J = K // 2
        while J >= 1:
            _pair_stage(ref, J, K)
            J //= 2
        _vreg_sort_pass(ref, K)
        K *= 2


def _sc_sort_ssd(px_hbm, py_hbm, out_hbm, vx, vy, vacc):
    wid = lax.axis_index("s") * _NC + lax.axis_index("c")
    b = wid // 4
    l0 = (wid % 4) * _IPW

    def item_body(i, acc):
        row = b * _LP + l0 + i
        pltpu.sync_copy(px_hbm.at[row], vx)
        pltpu.sync_copy(py_hbm.at[row], vy)
        _sort2048(vx)
        _sort2048(vy)

        def diff_body(v, a):
            d = vx[pl.ds(v * 16, 16)] - vy[pl.ds(v * 16, 16)]
            return a + d * d
        return lax.fori_loop(0, _NV, diff_body, acc, unroll=4)

    acc = lax.fori_loop(0, _IPW, item_body, jnp.zeros((16,), jnp.float32))
    vacc[...] = acc
    pltpu.sync_copy(vacc, out_hbm.at[wid])


def _final_kernel(part_ref, out_ref):
    v = part_ref[...]
    tot = jnp.sum(v, axis=1, keepdims=True)
    sw = jnp.sqrt(tot / (_N * _L))
    out_ref[0, 0] = jnp.sum(sw) / v.shape[0]


def kernel(X, Y, Pxy, Pyx, thetas):
    B = X.shape[0]
    th_p = jnp.pad(thetas, ((0, 0), (0, _LP - _L), (0, 0)))

    px, py = pl.pallas_call(
        _proj_kernel,
        grid=(B,),
        in_specs=[
            pl.BlockSpec((1, _N, _D), lambda b: (b, 0, 0)),
            pl.BlockSpec((1, _N, _D), lambda b: (b, 0, 0)),
            pl.BlockSpec((1, _LP, _D), lambda b: (b, 0, 0)),
        ],
        out_specs=[
            pl.BlockSpec((1, _LP, _N), lambda b: (b, 0, 0)),
            pl.BlockSpec((1, _LP, _N), lambda b: (b, 0, 0)),
        ],
        out_shape=[
            jax.ShapeDtypeStruct((B, _LP, _N), jnp.float32),
            jax.ShapeDtypeStruct((B, _LP, _N), jnp.float32),
        ],
    )(X, Y, th_p)

    mesh = plsc.VectorSubcoreMesh(
        core_axis_name="c", subcore_axis_name="s",
        num_cores=_NC, num_subcores=_NS)
    sc_call = pl.kernel(
        _sc_sort_ssd,
        mesh=mesh,
        out_type=jax.ShapeDtypeStruct((_NW, 16), jnp.float32),
        compiler_params=pltpu.CompilerParams(needs_layout_passes=False),
        scratch_types=[
            pltpu.VMEM((_N,), jnp.float32),
            pltpu.VMEM((_N,), jnp.float32),
            pltpu.VMEM((16,), jnp.float32),
        ],
    )
    part = sc_call(px.reshape(B * _LP, _N), py.reshape(B * _LP, _N))

    out = pl.pallas_call(
        _final_kernel,
        in_specs=[pl.BlockSpec((B, 4 * 16), lambda: (0, 0))],
        out_specs=pl.BlockSpec(memory_space=pltpu.SMEM),
        out_shape=jax.ShapeDtypeStruct((1, 1), jnp.float32),
    )(part.reshape(B, 4 * 16))

    return out[0, 0]


# speedup vs baseline: 3.1799x; 2.5023x over previous
import jax
import jax.numpy as jnp
from jax import lax
from jax.experimental import pallas as pl
from jax.experimental.pallas import tpu as pltpu
from jax.experimental.pallas import tpu_sc as plsc

_N = 2048
_D = 128
_L = 100
_NV = _N // 16
_NC = 2
_NS = 16
_NW = _NC * _NS
_LTC = 64
_LSC = _L - _LTC
_LSCP = 40
_IPW = 8 * _LSC // _NW


def _roll_up(x, j):
    return jnp.concatenate([x[j:], x[:j]], axis=0)


def _roll_down(x, j):
    return jnp.concatenate([x[-j:], x[:-j]], axis=0)


def _tc_sort_kernel(x_ref, y_ref, th_ref, out_ref):
    th = th_ref[0]
    norm2 = jnp.sum(th * th, axis=1, keepdims=True)
    tn = th / jnp.sqrt(norm2)
    dn = (((1,), (1,)), ((), ()))
    px = lax.dot_general(x_ref[0], tn, dn,
                         preferred_element_type=jnp.float32)
    py = lax.dot_general(y_ref[0], tn, dn,
                         preferred_element_type=jnp.float32)

    s = jnp.concatenate([px, py], axis=1)
    row = lax.broadcasted_iota(jnp.int32, (_N, 1), 0)
    k = 2
    while k <= _N:
        j = k // 2
        while j >= 1:
            up = (row & k) == 0
            low = (row & j) == 0
            part = jnp.where(low, _roll_up(s, j), _roll_down(s, j))
            mn = jnp.minimum(s, part)
            mx = jnp.maximum(s, part)
            s = jnp.where(up == low, mn, mx)
            j //= 2
        k *= 2

    d = s[:, :_LTC] - s[:, _LTC:]
    ssd = jnp.sum(d * d, axis=0, keepdims=True)
    out_ref[0] = jnp.concatenate(
        [ssd, jnp.zeros((1, 128 - _LTC), jnp.float32)], axis=1)


def _sc_proj_kernel(x_ref, y_ref, th_ref, px_ref, py_ref):
    th = th_ref[0]
    norm2 = jnp.sum(th * th, axis=1, keepdims=True)
    tn = th / jnp.sqrt(norm2)
    dn = (((1,), (1,)), ((), ()))
    px_ref[0] = lax.dot_general(tn, x_ref[0], dn,
                                preferred_element_type=jnp.float32)
    py_ref[0] = lax.dot_general(tn, y_ref[0], dn,
                                preferred_element_type=jnp.float32)


def _vreg_sort_pass(ref, K):
    if K >= _NV:
        def body(v, c):
            x = ref[pl.ds(v * 16, 16)]
            s, _ = plsc.sort_key_val(x, x)
            ref[pl.ds(v * 16, 16)] = s
            return c
        lax.fori_loop(0, _NV, body, 0, unroll=8)
        return

    def mk_body(off, desc):
        def body(q, c):
            v = (q // K) * (2 * K) + (q % K) + off
            x = ref[pl.ds(v * 16, 16)]
            s, _ = plsc.sort_key_val(x, x, descending=desc)
            ref[pl.ds(v * 16, 16)] = s
            return c
        return body
    lax.fori_loop(0, _NV // 2, mk_body(0, False), 0, unroll=8)
    lax.fori_loop(0, _NV // 2, mk_body(K, True), 0, unroll=8)


def _pair_stage(ref, J, K):
    H = K // 2

    def mk_body(off, desc):
        def body(q, c):
            if H >= _NV // 2:
                p = q
            else:
                p = (q // H) * (2 * H) + (q % H) + off
            v = (p // J) * (2 * J) + (p % J)
            w = v + J
            a = ref[pl.ds(v * 16, 16)]
            b = ref[pl.ds(w * 16, 16)]
            mn = jnp.minimum(a, b)
            mx = jnp.maximum(a, b)
            if desc:
                mn, mx = mx, mn
            ref[pl.ds(v * 16, 16)] = mn
            ref[pl.ds(w * 16, 16)] = mx
            return c
        return body

    if H >= _NV // 2:
        lax.fori_loop(0, _NV // 2, mk_body(0, False), 0, unroll=8)
    else:
        lax.fori_loop(0, _NV // 4, mk_body(0, False), 0, unroll=8)
        lax.fori_loop(0, _NV // 4, mk_body(H, True), 0, unroll=8)


def _sort2048(ref):
    _vreg_sort_pass(ref, 1)
    K = 2
    while K <= _NV:
        J = K // 2
        while J >= 1:
            _pair_stage(ref, J, K)
            J //= 2
        _vreg_sort_pass(ref, K)
        K *= 2


def _sc_sort_ssd(px_hbm, py_hbm, out_hbm, vx, vy, vacc):
    wid = lax.axis_index("s") * _NC + lax.axis_index("c")
    b = wid // 4
    l0 = (wid % 4) * _IPW

    def item_body(i, acc):
        row = b * _LSCP + l0 + i
        pltpu.sync_copy(px_hbm.at[row], vx)
        pltpu.sync_copy(py_hbm.at[row], vy)
        _sort2048(vx)
        _sort2048(vy)

        def diff_body(v, a):
            d = vx[pl.ds(v * 16, 16)] - vy[pl.ds(v * 16, 16)]
            return a + d * d
        return lax.fori_loop(0, _NV, diff_body, acc, unroll=4)

    acc = lax.fori_loop(0, _IPW, item_body, jnp.zeros((16,), jnp.float32))
    vacc[...] = acc
    pltpu.sync_copy(vacc, out_hbm.at[wid])


def _final_kernel(tc_ref, sc_ref, out_ref):
    tc = tc_ref[...]
    sc = sc_ref[...]
    tot = (jnp.sum(tc, axis=1, keepdims=True)
           + jnp.sum(sc, axis=1, keepdims=True))
    sw = jnp.sqrt(tot / (_N * _L))
    out_ref[0, 0] = jnp.sum(sw) / tc.shape[0]


def kernel(X, Y, Pxy, Pyx, thetas):
    B = X.shape[0]
    th_tc = thetas[:, :_LTC, :]
    th_sc = jnp.pad(thetas[:, _LTC:, :], ((0, 0), (0, _LSCP - _LSC), (0, 0)))

    px, py = pl.pallas_call(
        _sc_proj_kernel,
        grid=(B,),
        in_specs=[
            pl.BlockSpec((1, _N, _D), lambda b: (b, 0, 0)),
            pl.BlockSpec((1, _N, _D), lambda b: (b, 0, 0)),
            pl.BlockSpec((1, _LSCP, _D), lambda b: (b, 0, 0)),
        ],
        out_specs=[
            pl.BlockSpec((1, _LSCP, _N), lambda b: (b, 0, 0)),
            pl.BlockSpec((1, _LSCP, _N), lambda b: (b, 0, 0)),
        ],
        out_shape=[
            jax.ShapeDtypeStruct((B, _LSCP, _N), jnp.float32),
            jax.ShapeDtypeStruct((B, _LSCP, _N), jnp.float32),
        ],
    )(X, Y, th_sc)

    mesh = plsc.VectorSubcoreMesh(
        core_axis_name="c", subcore_axis_name="s",
        num_cores=_NC, num_subcores=_NS)
    sc_part = pl.kernel(
        _sc_sort_ssd,
        mesh=mesh,
        out_type=jax.ShapeDtypeStruct((_NW, 16), jnp.float32),
        compiler_params=pltpu.CompilerParams(needs_layout_passes=False),
        scratch_types=[
            pltpu.VMEM((_N,), jnp.float32),
            pltpu.VMEM((_N,), jnp.float32),
            pltpu.VMEM((16,), jnp.float32),
        ],
    )(px.reshape(B * _LSCP, _N), py.reshape(B * _LSCP, _N))

    tc_ssd = pl.pallas_call(
        _tc_sort_kernel,
        grid=(B,),
        in_specs=[
            pl.BlockSpec((1, _N, _D), lambda b: (b, 0, 0)),
            pl.BlockSpec((1, _N, _D), lambda b: (b, 0, 0)),
            pl.BlockSpec((1, _LTC, _D), lambda b: (b, 0, 0)),
        ],
        out_specs=pl.BlockSpec((1, 1, 128), lambda b: (b, 0, 0)),
        out_shape=jax.ShapeDtypeStruct((B, 1, 128), jnp.float32),
    )(X, Y, th_tc)

    out = pl.pallas_call(
        _final_kernel,
        in_specs=[
            pl.BlockSpec((B, 128), lambda: (0, 0)),
            pl.BlockSpec((B, 4 * 16), lambda: (0, 0)),
        ],
        out_specs=pl.BlockSpec(memory_space=pltpu.SMEM),
        out_shape=jax.ShapeDtypeStruct((1, 1), jnp.float32),
    )(tc_ssd.reshape(B, 128), sc_part.reshape(B, 4 * 16))

    return out[0, 0]


# speedup vs baseline: 3.1981x; 1.0057x over previous
import jax
import jax.numpy as jnp
from jax import lax
from jax.experimental import pallas as pl
from jax.experimental.pallas import tpu as pltpu
from jax.experimental.pallas import tpu_sc as plsc

_N = 2048
_D = 128
_L = 100
_NV = _N // 16
_NC = 2
_NS = 16
_NW = _NC * _NS
_LTC = 64
_LSC = _L - _LTC
_LSCP = 40
_IPW = 8 * _LSC // _NW


def _roll_up(x, j):
    return jnp.concatenate([x[j:], x[:j]], axis=0)


def _roll_down(x, j):
    return jnp.concatenate([x[-j:], x[:-j]], axis=0)


def _tc_sort_kernel(x_ref, y_ref, th_ref, out_ref):
    th = th_ref[0]
    norm2 = jnp.sum(th * th, axis=1, keepdims=True)
    tn = th / jnp.sqrt(norm2)
    dn = (((1,), (1,)), ((), ()))
    px = lax.dot_general(x_ref[0], tn, dn,
                         preferred_element_type=jnp.float32)
    py = lax.dot_general(y_ref[0], tn, dn,
                         preferred_element_type=jnp.float32)

    s = jnp.concatenate([px, py], axis=1)
    row = lax.broadcasted_iota(jnp.int32, (_N, 1), 0)
    k = 2
    while k <= _N:
        j = k // 2
        while j >= 1:
            if j >= 8:
                pieces = []
                for a in range(0, _N, 2 * j):
                    blk_a = s[a:a + j]
                    blk_b = s[a + j:a + 2 * j]
                    mn = jnp.minimum(blk_a, blk_b)
                    mx = jnp.maximum(blk_a, blk_b)
                    pieces += [mn, mx] if (a & k) == 0 else [mx, mn]
                s = jnp.concatenate(pieces, axis=0)
            else:
                up = (row & k) == 0
                low = (row & j) == 0
                part = jnp.where(low, _roll_up(s, j), _roll_down(s, j))
                mn = jnp.minimum(s, part)
                mx = jnp.maximum(s, part)
                s = jnp.where(up == low, mn, mx)
            j //= 2
        k *= 2

    d = s[:, :_LTC] - s[:, _LTC:]
    ssd = jnp.sum(d * d, axis=0, keepdims=True)
    out_ref[0] = jnp.concatenate(
        [ssd, jnp.zeros((1, 128 - _LTC), jnp.float32)], axis=1)


def _sc_proj_kernel(x_ref, y_ref, th_ref, px_ref, py_ref):
    th = th_ref[0]
    norm2 = jnp.sum(th * th, axis=1, keepdims=True)
    tn = th / jnp.sqrt(norm2)
    dn = (((1,), (1,)), ((), ()))
    px_ref[0] = lax.dot_general(tn, x_ref[0], dn,
                                preferred_element_type=jnp.float32)
    py_ref[0] = lax.dot_general(tn, y_ref[0], dn,
                                preferred_element_type=jnp.float32)


def _vreg_sort_pass(ref, K):
    if K >= _NV:
        def body(v, c):
            x = ref[pl.ds(v * 16, 16)]
            s, _ = plsc.sort_key_val(x, x)
            ref[pl.ds(v * 16, 16)] = s
            return c
        lax.fori_loop(0, _NV, body, 0, unroll=8)
        return

    def mk_body(off, desc):
        def body(q, c):
            v = (q // K) * (2 * K) + (q % K) + off
            x = ref[pl.ds(v * 16, 16)]
            s, _ = plsc.sort_key_val(x, x, descending=desc)
            ref[pl.ds(v * 16, 16)] = s
            return c
        return body
    lax.fori_loop(0, _NV // 2, mk_body(0, False), 0, unroll=8)
    lax.fori_loop(0, _NV // 2, mk_body(K, True), 0, unroll=8)


def _pair_stage(ref, J, K):
    H = K // 2

    def mk_body(off, desc):
        def body(q, c):
            if H >= _NV // 2:
                p = q
            else:
                p = (q // H) * (2 * H) + (q % H) + off
            v = (p // J) * (2 * J) + (p % J)
            w = v + J
            a = ref[pl.ds(v * 16, 16)]
            b = ref[pl.ds(w * 16, 16)]
            mn = jnp.minimum(a, b)
            mx = jnp.maximum(a, b)
            if desc:
                mn, mx = mx, mn
            ref[pl.ds(v * 16, 16)] = mn
            ref[pl.ds(w * 16, 16)] = mx
            return c
        return body

    if H >= _NV // 2:
        lax.fori_loop(0, _NV // 2, mk_body(0, False), 0, unroll=8)
    else:
        lax.fori_loop(0, _NV // 4, mk_body(0, False), 0, unroll=8)
        lax.fori_loop(0, _NV // 4, mk_body(H, True), 0, unroll=8)


_PROBE_SKIP_VSORT = False
_PROBE_SKIP_PAIRS = False


def _sort2048(ref):
    if not _PROBE_SKIP_VSORT:
        _vreg_sort_pass(ref, 1)
    K = 2
    while K <= _NV:
        J = K // 2
        while J >= 1:
            if not _PROBE_SKIP_PAIRS:
                _pair_stage(ref, J, K)
            J //= 2
        if not _PROBE_SKIP_VSORT:
            _vreg_sort_pass(ref, K)
        K *= 2


def _sc_sort_ssd(px_hbm, py_hbm, out_hbm, vx, vy, vacc):
    wid = lax.axis_index("s") * _NC + lax.axis_index("c")
    b = wid // 4
    l0 = (wid % 4) * _IPW

    def item_body(i, acc):
        row = b * _LSCP + l0 + i
        pltpu.sync_copy(px_hbm.at[row], vx)
        pltpu.sync_copy(py_hbm.at[row], vy)
        _sort2048(vx)
        _sort2048(vy)

        def diff_body(v, a):
            d = vx[pl.ds(v * 16, 16)] - vy[pl.ds(v * 16, 16)]
            return a + d * d
        return lax.fori_loop(0, _NV, diff_body, acc, unroll=4)

    acc = lax.fori_loop(0, _IPW, item_body, jnp.zeros((16,), jnp.float32))
    vacc[...] = acc
    pltpu.sync_copy(vacc, out_hbm.at[wid])


def _final_kernel(tc_ref, sc_ref, out_ref):
    tc = tc_ref[...]
    sc = sc_ref[...]
    tot = (jnp.sum(tc, axis=1, keepdims=True)
           + jnp.sum(sc, axis=1, keepdims=True))
    sw = jnp.sqrt(tot / (_N * _L))
    out_ref[0, 0] = jnp.sum(sw) / tc.shape[0]


def kernel(X, Y, Pxy, Pyx, thetas):
    B = X.shape[0]
    th_tc = thetas[:, :_LTC, :]
    th_sc = jnp.pad(thetas[:, _LTC:, :], ((0, 0), (0, _LSCP - _LSC), (0, 0)))

    px, py = pl.pallas_call(
        _sc_proj_kernel,
        grid=(B,),
        in_specs=[
            pl.BlockSpec((1, _N, _D), lambda b: (b, 0, 0)),
            pl.BlockSpec((1, _N, _D), lambda b: (b, 0, 0)),
            pl.BlockSpec((1, _LSCP, _D), lambda b: (b, 0, 0)),
        ],
        out_specs=[
            pl.BlockSpec((1, _LSCP, _N), lambda b: (b, 0, 0)),
            pl.BlockSpec((1, _LSCP, _N), lambda b: (b, 0, 0)),
        ],
        out_shape=[
            jax.ShapeDtypeStruct((B, _LSCP, _N), jnp.float32),
            jax.ShapeDtypeStruct((B, _LSCP, _N), jnp.float32),
        ],
    )(X, Y, th_sc)

    mesh = plsc.VectorSubcoreMesh(
        core_axis_name="c", subcore_axis_name="s",
        num_cores=_NC, num_subcores=_NS)
    sc_part = pl.kernel(
        _sc_sort_ssd,
        mesh=mesh,
        out_type=jax.ShapeDtypeStruct((_NW, 16), jnp.float32),
        compiler_params=pltpu.CompilerParams(needs_layout_passes=False),
        scratch_types=[
            pltpu.VMEM((_N,), jnp.float32),
            pltpu.VMEM((_N,), jnp.float32),
            pltpu.VMEM((16,), jnp.float32),
        ],
    )(px.reshape(B * _LSCP, _N), py.reshape(B * _LSCP, _N))

    tc_ssd = pl.pallas_call(
        _tc_sort_kernel,
        grid=(B,),
        in_specs=[
            pl.BlockSpec((1, _N, _D), lambda b: (b, 0, 0)),
            pl.BlockSpec((1, _N, _D), lambda b: (b, 0, 0)),
            pl.BlockSpec((1, _LTC, _D), lambda b: (b, 0, 0)),
        ],
        out_specs=pl.BlockSpec((1, 1, 128), lambda b: (b, 0, 0)),
        out_shape=jax.ShapeDtypeStruct((B, 1, 128), jnp.float32),
    )(X, Y, th_tc)

    out = pl.pallas_call(
        _final_kernel,
        in_specs=[
            pl.BlockSpec((B, 128), lambda: (0, 0)),
            pl.BlockSpec((B, 4 * 16), lambda: (0, 0)),
        ],
        out_specs=pl.BlockSpec(memory_space=pltpu.SMEM),
        out_shape=jax.ShapeDtypeStruct((1, 1), jnp.float32),
    )(tc_ssd.reshape(B, 128), sc_part.reshape(B, 4 * 16))

    return out[0, 0]
